# baseline (speedup 1.0000x reference)
import functools

import jax
import jax.numpy as jnp
from jax import lax
from jax.experimental import pallas as pl
from jax.experimental.pallas import tpu as pltpu
from jax.experimental.pallas import tpu_sc as plsc


def kernel(x, exe_ids, pe):
    S, B, D = x.shape
    info = plsc.get_sparse_core_info()
    NC, NS, L = info.num_cores, info.num_subcores, info.num_lanes
    NW = NC * NS
    per_w = S // NW
    CH = 16
    mesh = plsc.VectorSubcoreMesh(core_axis_name="c", subcore_axis_name="s")

    @functools.partial(
        pl.kernel,
        mesh=mesh,
        out_type=jax.ShapeDtypeStruct((S, B, D), jnp.float32),
        scratch_types=[
            pltpu.VMEM((CH, B, D), jnp.float32),
            pltpu.VMEM((CH, 1, D), jnp.float32),
            pltpu.VMEM((CH * B,), jnp.int32),
        ],
    )
    def k(x_hbm, e_hbm, pe_hbm, o_hbm, xv, pev, ev):
        wid = lax.axis_index("s") * NC + lax.axis_index("c")
        base = wid * per_w
        for c in range(per_w // CH):
            s0 = base + c * CH
            pltpu.sync_copy(x_hbm.at[pl.ds(s0, CH)], xv)
            pltpu.sync_copy(pe_hbm.at[pl.ds(s0, CH)], pev)
            pltpu.sync_copy(e_hbm.at[pl.ds(s0 * B, CH * B)], ev)

            def dloop(j, carry):
                sl = pl.ds(j * L, L)
                for si in range(CH):
                    pvec = pev[si, 0, sl]
                    for b in range(B):
                        f = si * B + b
                        e_vec = ev[pl.ds((f // L) * L, L)]
                        scale = jnp.where(e_vec[f % L] != 0, 2.0, 1.0)
                        xv[si, b, sl] = xv[si, b, sl] + pvec * scale
                return carry

            lax.fori_loop(0, D // L, dloop, 0)
            pltpu.sync_copy(xv, o_hbm.at[pl.ds(s0, CH)])

    return k(x, exe_ids.reshape(S * B), pe)


# speedup vs baseline: 11.2316x; 11.2316x over previous
import jax
import jax.numpy as jnp
from jax.experimental import pallas as pl

_BS = 512
_BD = 512


def _body(x_ref, e_ref, pe_ref, o_ref):
    scale = jnp.where(e_ref[...] != 0, 2.0, 1.0)
    o_ref[...] = x_ref[...] + pe_ref[...] * scale[:, :, None]


def kernel(x, exe_ids, pe):
    S, B, D = x.shape
    BS, BD = _BS, _BD
    grid = (S // BS, D // BD)
    return pl.pallas_call(
        _body,
        grid=grid,
        in_specs=[
            pl.BlockSpec((BS, B, BD), lambda i, j: (i, 0, j)),
            pl.BlockSpec((BS, B), lambda i, j: (i, 0)),
            pl.BlockSpec((BS, 1, BD), lambda i, j: (i, 0, j)),
        ],
        out_specs=pl.BlockSpec((BS, B, BD), lambda i, j: (i, 0, j)),
        out_shape=jax.ShapeDtypeStruct(x.shape, x.dtype),
    )(x, exe_ids, pe)
